# baseline (speedup 1.0000x reference)
import functools

import jax
import jax.numpy as jnp
from jax import lax
from jax.experimental import pallas as pl
from jax.experimental.pallas import tpu as pltpu
from jax.experimental.pallas import tpu_sc as plsc

NC = 2
NS = 16
NW = NC * NS
LANES = 16
CHUNK = 128


def _sc_mesh():
    return plsc.VectorSubcoreMesh(
        core_axis_name="c", subcore_axis_name="s", num_cores=NC, num_subcores=NS
    )


def _make_repack_kernel(e, n, n_pad, ept):
    def body(ei_ref, src_ref, dst_ref):
        w = pl.program_id(0)
        ei = ei_ref[...]
        pos = w * ept + lax.broadcasted_iota(jnp.int32, (1, ept), 1)
        valid = pos < e
        src_o = jnp.where(valid, ei[0:1, :], pos % n)
        dst_o = jnp.where(valid, ei[1:2, :], n + pos % (n_pad - n))
        src_ref[...] = src_o.reshape(1, 1, ept)
        dst_ref[...] = dst_o.reshape(1, 1, ept)

    return pl.pallas_call(
        body,
        grid=(NW,),
        in_specs=[pl.BlockSpec((2, ept), lambda w: (0, w))],
        out_specs=[
            pl.BlockSpec((1, 1, ept), lambda w: (w, 0, 0)),
            pl.BlockSpec((1, 1, ept), lambda w: (w, 0, 0)),
        ],
        out_shape=[
            jax.ShapeDtypeStruct((NW, 1, ept), jnp.int32),
            jax.ShapeDtypeStruct((NW, 1, ept), jnp.int32),
        ],
    )


def _make_deg_kernel(n_pad, ept):
    @functools.partial(
        pl.kernel,
        out_type=jax.ShapeDtypeStruct((NW, n_pad), jnp.float32),
        mesh=_sc_mesh(),
        compiler_params=pltpu.CompilerParams(needs_layout_passes=False),
        scratch_types=[
            pltpu.VMEM((ept,), jnp.int32),
            pltpu.VMEM((n_pad,), jnp.float32),
        ],
    )
    def deg_kernel(dst_hbm, degp_hbm, idx_v, cnt_v):
        cid = lax.axis_index("c")
        sid = lax.axis_index("s")
        wid = cid * NS + sid
        zeros = jnp.zeros((LANES,), jnp.float32)

        @pl.loop(0, n_pad // LANES)
        def _(i):
            cnt_v[pl.ds(i * LANES, LANES)] = zeros

        pltpu.sync_copy(dst_hbm.at[wid, 0], idx_v)
        ones = jnp.ones((LANES,), jnp.float32)

        @pl.loop(0, ept // LANES)
        def _(i):
            idx = idx_v[pl.ds(i * LANES, LANES)]
            plsc.addupdate_scatter(cnt_v, [idx], ones)

        pltpu.sync_copy(cnt_v, degp_hbm.at[wid])

    return deg_kernel


def _make_scatter_kernel(n, d, n_pad, cpt):
    z_rows_per_tile = n_pad // NS
    ept = cpt * CHUNK

    @functools.partial(
        pl.kernel,
        out_type=jax.ShapeDtypeStruct((NC, n_pad, d), jnp.float32),
        mesh=_sc_mesh(),
        scratch_types=[
            pltpu.VMEM((ept,), jnp.int32),
            pltpu.VMEM((2, CHUNK), jnp.int32),
            pltpu.VMEM((CHUNK, d), jnp.float32),
            pltpu.VMEM((CHUNK, d), jnp.float32),
            pltpu.VMEM_SHARED((n_pad, d), jnp.float32),
            pltpu.SemaphoreType.DMA,
            pltpu.SemaphoreType.DMA,
            pltpu.SemaphoreType.DMA,
            pltpu.SemaphoreType.DMA,
        ],
    )
    def scatter_kernel(y_hbm, src_hbm, dst_hbm, z_hbm,
                       src_v, dstr, buf0, buf1, z_sh,
                       gsem0, gsem1, dsem0, dsem1):
        cid = lax.axis_index("c")
        sid = lax.axis_index("s")
        wid = cid * NS + sid

        pltpu.sync_copy(src_hbm.at[wid, 0], src_v)

        zeros = jnp.zeros((LANES,), jnp.float32)

        @pl.loop(0, CHUNK)
        def _(r):
            for k in range(d // LANES):
                buf0[r, pl.ds(k * LANES, LANES)] = zeros

        base = sid * z_rows_per_tile
        for k in range(z_rows_per_tile // CHUNK):
            pltpu.sync_copy(buf0, z_sh.at[pl.ds(base + k * CHUNK, CHUNK)])
        plsc.subcore_barrier()

        bufs = (buf0, buf1)
        gsems = (gsem0, gsem1)
        dsems = (dsem0, dsem1)
        for s in (0, 1):
            pltpu.async_copy(dst_hbm.at[wid, 0, pl.ds(s * CHUNK, CHUNK)],
                             dstr.at[s], dsems[s])
            pltpu.async_copy(y_hbm.at[src_v.at[pl.ds(s * CHUNK, CHUNK)]],
                             bufs[s], gsems[s])

        @pl.loop(0, cpt, step=2)
        def _(jo):
            for b in range(2):
                j = jo + b
                pltpu.make_async_copy(dst_hbm.at[wid, 0, pl.ds(0, CHUNK)],
                                      dstr.at[b], dsems[b]).wait()
                pltpu.make_async_copy(y_hbm.at[src_v.at[pl.ds(0, CHUNK)]],
                                      bufs[b], gsems[b]).wait()
                pltpu.sync_copy(bufs[b], z_sh.at[dstr.at[b]], add=True)

                @pl.when(j + 2 < cpt)
                def _():
                    pltpu.async_copy(
                        dst_hbm.at[wid, 0, pl.ds((j + 2) * CHUNK, CHUNK)],
                        dstr.at[b], dsems[b])
                    pltpu.async_copy(
                        y_hbm.at[src_v.at[pl.ds((j + 2) * CHUNK, CHUNK)]],
                        bufs[b], gsems[b])

        plsc.subcore_barrier()
        pltpu.sync_copy(z_sh.at[pl.ds(base, z_rows_per_tile)],
                        z_hbm.at[cid, pl.ds(base, z_rows_per_tile)])

    return scatter_kernel


def _make_prep_kernel(n, d, n_pad):
    def body(x_ref, w_ref, degp_ref, y_ref, dinv_ref):
        cnt = jnp.sum(degp_ref[...], axis=0)[:n]
        dinv = lax.rsqrt(cnt + 1.0)
        xw = jnp.dot(x_ref[...], w_ref[...], preferred_element_type=jnp.float32)
        y_ref[...] = xw * dinv[:, None]
        dinv_ref[...] = dinv[:, None]

    return pl.pallas_call(
        body,
        grid=(1,),
        in_specs=[
            pl.BlockSpec((n, d), lambda i: (0, 0)),
            pl.BlockSpec((d, d), lambda i: (0, 0)),
            pl.BlockSpec((NW, n_pad), lambda i: (0, 0)),
        ],
        out_specs=[
            pl.BlockSpec((n, d), lambda i: (0, 0)),
            pl.BlockSpec((n, 1), lambda i: (0, 0)),
        ],
        out_shape=[
            jax.ShapeDtypeStruct((n, d), jnp.float32),
            jax.ShapeDtypeStruct((n, 1), jnp.float32),
        ],
    )


def _make_head_kernel(n, d):
    def body(z_ref, y_ref, dinv_ref, bc_ref, w1_ref, b1_ref, w2_ref, b2_ref,
             o_ref):
        zsum = z_ref[0] + z_ref[1] + y_ref[...]
        h = jnp.maximum(zsum * dinv_ref[...] + bc_ref[...], 0.0)
        h = jnp.maximum(
            jnp.dot(h, w1_ref[...], preferred_element_type=jnp.float32)
            + b1_ref[...], 0.0)
        h = jnp.maximum(
            jnp.dot(h, w2_ref[...], preferred_element_type=jnp.float32)
            + b2_ref[...], 0.0)
        o_ref[...] = jax.nn.sigmoid(h)

    return pl.pallas_call(
        body,
        grid=(1,),
        in_specs=[
            pl.BlockSpec((NC, n, d), lambda i: (0, 0, 0)),
            pl.BlockSpec((n, d), lambda i: (0, 0)),
            pl.BlockSpec((n, 1), lambda i: (0, 0)),
            pl.BlockSpec((1, d), lambda i: (0, 0)),
            pl.BlockSpec((d, d), lambda i: (0, 0)),
            pl.BlockSpec((1, d), lambda i: (0, 0)),
            pl.BlockSpec((d, 1), lambda i: (0, 0)),
            pl.BlockSpec((1, 1), lambda i: (0, 0)),
        ],
        out_specs=pl.BlockSpec((n, 1), lambda i: (0, 0)),
        out_shape=jax.ShapeDtypeStruct((n, 1), jnp.float32),
    )


def kernel(x, edge_index, W_conv, b_conv, W_lin1, b_lin1, W_lin2, b_lin2):
    n, d = x.shape
    e = edge_index.shape[1]

    cpt = pl.cdiv(e, NW * CHUNK)
    cpt = cpt + (cpt % 2)
    ept = cpt * CHUNK
    n_pad = ((n + NS * CHUNK - 1) // (NS * CHUNK)) * (NS * CHUNK)
    if n_pad == n:
        n_pad += NS * CHUNK

    srcp, dstp = _make_repack_kernel(e, n, n_pad, ept)(edge_index)
    degp = _make_deg_kernel(n_pad, ept)(dstp)
    y, dinv = _make_prep_kernel(n, d, n_pad)(x, W_conv, degp)
    z = _make_scatter_kernel(n, d, n_pad, cpt)(y, srcp, dstp)
    out = _make_head_kernel(n, d)(
        z, y, dinv, b_conv.reshape(1, d), W_lin1, b_lin1.reshape(1, d),
        W_lin2, b_lin2.reshape(1, 1))
    return out


# speedup vs baseline: 1.0307x; 1.0307x over previous
import functools

import jax
import jax.numpy as jnp
from jax import lax
from jax.experimental import pallas as pl
from jax.experimental.pallas import tpu as pltpu
from jax.experimental.pallas import tpu_sc as plsc

NC = 2
NS = 16
NW = NC * NS
LANES = 16
CHUNK = 128


def _sc_mesh():
    return plsc.VectorSubcoreMesh(
        core_axis_name="c", subcore_axis_name="s", num_cores=NC, num_subcores=NS
    )


def _make_repack_kernel(e, n, n_pad, ept):
    smask = (1 << (n.bit_length() - 1)) - 1
    tmask = (1 << ((n_pad - n).bit_length() - 1)) - 1
    last_full = e // ept

    def body(ei_ref, src_ref, dst_ref):
        w = pl.program_id(0)
        ei = ei_ref[...]

        @pl.when(w < last_full)
        def _():
            src_ref[...] = ei[0:1, :].reshape(1, 1, ept)
            dst_ref[...] = ei[1:2, :].reshape(1, 1, ept)

        @pl.when(w >= last_full)
        def _():
            pos = w * ept + lax.broadcasted_iota(jnp.int32, (1, ept), 1)
            valid = pos < e
            src_o = jnp.where(valid, ei[0:1, :], pos & smask)
            dst_o = jnp.where(valid, ei[1:2, :], n + (pos & tmask))
            src_ref[...] = src_o.reshape(1, 1, ept)
            dst_ref[...] = dst_o.reshape(1, 1, ept)

    return pl.pallas_call(
        body,
        grid=(NW,),
        in_specs=[pl.BlockSpec((2, ept), lambda w: (0, w))],
        out_specs=[
            pl.BlockSpec((1, 1, ept), lambda w: (w, 0, 0)),
            pl.BlockSpec((1, 1, ept), lambda w: (w, 0, 0)),
        ],
        out_shape=[
            jax.ShapeDtypeStruct((NW, 1, ept), jnp.int32),
            jax.ShapeDtypeStruct((NW, 1, ept), jnp.int32),
        ],
    )


def _make_deg_kernel(n_pad, ept):
    @functools.partial(
        pl.kernel,
        out_type=jax.ShapeDtypeStruct((NW, n_pad), jnp.float32),
        mesh=_sc_mesh(),
        compiler_params=pltpu.CompilerParams(needs_layout_passes=False),
        scratch_types=[
            pltpu.VMEM((ept,), jnp.int32),
            pltpu.VMEM((n_pad,), jnp.float32),
        ],
    )
    def deg_kernel(dst_hbm, degp_hbm, idx_v, cnt_v):
        cid = lax.axis_index("c")
        sid = lax.axis_index("s")
        wid = cid * NS + sid
        zeros = jnp.zeros((LANES,), jnp.float32)

        @pl.loop(0, n_pad // LANES)
        def _(i):
            cnt_v[pl.ds(i * LANES, LANES)] = zeros

        pltpu.sync_copy(dst_hbm.at[wid, 0], idx_v)
        ones = jnp.ones((LANES,), jnp.float32)

        @pl.loop(0, ept // LANES)
        def _(i):
            idx = idx_v[pl.ds(i * LANES, LANES)]
            plsc.addupdate_scatter(cnt_v, [idx], ones)

        pltpu.sync_copy(cnt_v, degp_hbm.at[wid])

    return deg_kernel


def _make_scatter_kernel(n, d, n_pad, cpt):
    z_rows_per_tile = n_pad // NS
    ept = cpt * CHUNK

    @functools.partial(
        pl.kernel,
        out_type=jax.ShapeDtypeStruct((NC, n_pad, d), jnp.float32),
        mesh=_sc_mesh(),
        scratch_types=[
            pltpu.VMEM((ept,), jnp.int32),
            pltpu.VMEM((2, CHUNK), jnp.int32),
            pltpu.VMEM((CHUNK, d), jnp.float32),
            pltpu.VMEM((CHUNK, d), jnp.float32),
            pltpu.VMEM_SHARED((n_pad, d), jnp.float32),
            pltpu.SemaphoreType.DMA,
            pltpu.SemaphoreType.DMA,
            pltpu.SemaphoreType.DMA,
            pltpu.SemaphoreType.DMA,
        ],
    )
    def scatter_kernel(y_hbm, src_hbm, dst_hbm, z_hbm,
                       src_v, dstr, buf0, buf1, z_sh,
                       gsem0, gsem1, dsem0, dsem1):
        cid = lax.axis_index("c")
        sid = lax.axis_index("s")
        wid = cid * NS + sid

        pltpu.sync_copy(src_hbm.at[wid, 0], src_v)

        zeros = jnp.zeros((LANES,), jnp.float32)

        @pl.loop(0, CHUNK)
        def _(r):
            for k in range(d // LANES):
                buf0[r, pl.ds(k * LANES, LANES)] = zeros

        base = sid * z_rows_per_tile
        for k in range(z_rows_per_tile // CHUNK):
            pltpu.sync_copy(buf0, z_sh.at[pl.ds(base + k * CHUNK, CHUNK)])
        plsc.subcore_barrier()

        bufs = (buf0, buf1)
        gsems = (gsem0, gsem1)
        dsems = (dsem0, dsem1)
        for s in (0, 1):
            pltpu.async_copy(dst_hbm.at[wid, 0, pl.ds(s * CHUNK, CHUNK)],
                             dstr.at[s], dsems[s])
            pltpu.async_copy(y_hbm.at[src_v.at[pl.ds(s * CHUNK, CHUNK)]],
                             bufs[s], gsems[s])

        @pl.loop(0, cpt, step=2)
        def _(jo):
            for b in range(2):
                j = jo + b
                pltpu.make_async_copy(dst_hbm.at[wid, 0, pl.ds(0, CHUNK)],
                                      dstr.at[b], dsems[b]).wait()
                pltpu.make_async_copy(y_hbm.at[src_v.at[pl.ds(0, CHUNK)]],
                                      bufs[b], gsems[b]).wait()
                pltpu.sync_copy(bufs[b], z_sh.at[dstr.at[b]], add=True)

                @pl.when(j + 2 < cpt)
                def _():
                    pltpu.async_copy(
                        dst_hbm.at[wid, 0, pl.ds((j + 2) * CHUNK, CHUNK)],
                        dstr.at[b], dsems[b])
                    pltpu.async_copy(
                        y_hbm.at[src_v.at[pl.ds((j + 2) * CHUNK, CHUNK)]],
                        bufs[b], gsems[b])

        plsc.subcore_barrier()
        pltpu.sync_copy(z_sh.at[pl.ds(base, z_rows_per_tile)],
                        z_hbm.at[cid, pl.ds(base, z_rows_per_tile)])

    return scatter_kernel


def _make_prep_kernel(n, d, n_pad):
    def body(x_ref, w_ref, degp_ref, y_ref, dinv_ref):
        cnt = jnp.sum(degp_ref[...], axis=0)[:n]
        dinv = lax.rsqrt(cnt + 1.0)
        xw = jnp.dot(x_ref[...], w_ref[...], preferred_element_type=jnp.float32)
        y_ref[...] = xw * dinv[:, None]
        dinv_ref[...] = dinv[:, None]

    return pl.pallas_call(
        body,
        grid=(1,),
        in_specs=[
            pl.BlockSpec((n, d), lambda i: (0, 0)),
            pl.BlockSpec((d, d), lambda i: (0, 0)),
            pl.BlockSpec((NW, n_pad), lambda i: (0, 0)),
        ],
        out_specs=[
            pl.BlockSpec((n, d), lambda i: (0, 0)),
            pl.BlockSpec((n, 1), lambda i: (0, 0)),
        ],
        out_shape=[
            jax.ShapeDtypeStruct((n, d), jnp.float32),
            jax.ShapeDtypeStruct((n, 1), jnp.float32),
        ],
    )


def _make_head_kernel(n, d):
    def body(z_ref, y_ref, dinv_ref, bc_ref, w1_ref, b1_ref, w2_ref, b2_ref,
             o_ref):
        zsum = z_ref[0] + z_ref[1] + y_ref[...]
        h = jnp.maximum(zsum * dinv_ref[...] + bc_ref[...], 0.0)
        h = jnp.maximum(
            jnp.dot(h, w1_ref[...], preferred_element_type=jnp.float32)
            + b1_ref[...], 0.0)
        h = jnp.maximum(
            jnp.dot(h, w2_ref[...], preferred_element_type=jnp.float32)
            + b2_ref[...], 0.0)
        o_ref[...] = jax.nn.sigmoid(h)

    return pl.pallas_call(
        body,
        grid=(1,),
        in_specs=[
            pl.BlockSpec((NC, n, d), lambda i: (0, 0, 0)),
            pl.BlockSpec((n, d), lambda i: (0, 0)),
            pl.BlockSpec((n, 1), lambda i: (0, 0)),
            pl.BlockSpec((1, d), lambda i: (0, 0)),
            pl.BlockSpec((d, d), lambda i: (0, 0)),
            pl.BlockSpec((1, d), lambda i: (0, 0)),
            pl.BlockSpec((d, 1), lambda i: (0, 0)),
            pl.BlockSpec((1, 1), lambda i: (0, 0)),
        ],
        out_specs=pl.BlockSpec((n, 1), lambda i: (0, 0)),
        out_shape=jax.ShapeDtypeStruct((n, 1), jnp.float32),
    )


def kernel(x, edge_index, W_conv, b_conv, W_lin1, b_lin1, W_lin2, b_lin2):
    n, d = x.shape
    e = edge_index.shape[1]

    cpt = pl.cdiv(e, NW * CHUNK)
    cpt = cpt + (cpt % 2)
    ept = cpt * CHUNK
    n_pad = ((n + NS * CHUNK - 1) // (NS * CHUNK)) * (NS * CHUNK)
    if n_pad == n:
        n_pad += NS * CHUNK

    srcp, dstp = _make_repack_kernel(e, n, n_pad, ept)(edge_index)
    degp = _make_deg_kernel(n_pad, ept)(dstp)
    y, dinv = _make_prep_kernel(n, d, n_pad)(x, W_conv, degp)
    z = _make_scatter_kernel(n, d, n_pad, cpt)(y, srcp, dstp)
    out = _make_head_kernel(n, d)(
        z, y, dinv, b_conv.reshape(1, d), W_lin1, b_lin1.reshape(1, d),
        W_lin2, b_lin2.reshape(1, 1))
    return out


# speedup vs baseline: 1.0431x; 1.0121x over previous
import functools

import jax
import jax.numpy as jnp
from jax import lax
from jax.experimental import pallas as pl
from jax.experimental.pallas import tpu as pltpu
from jax.experimental.pallas import tpu_sc as plsc

NC = 2
NS = 16
NW = NC * NS
LANES = 16
CHUNK = 128


def _sc_mesh():
    return plsc.VectorSubcoreMesh(
        core_axis_name="c", subcore_axis_name="s", num_cores=NC, num_subcores=NS
    )


def _make_repack_kernel(e, n, n_pad, ept):
    smask = (1 << (n.bit_length() - 1)) - 1
    tmask = (1 << ((n_pad - n).bit_length() - 1)) - 1
    last_full = e // ept

    def body(ei_ref, src_ref, dst_ref):
        w = pl.program_id(0)
        ei = ei_ref[...]

        @pl.when(w < last_full)
        def _():
            src_ref[...] = ei[0:1, :].reshape(1, 1, ept)
            dst_ref[...] = ei[1:2, :].reshape(1, 1, ept)

        @pl.when(w >= last_full)
        def _():
            pos = w * ept + lax.broadcasted_iota(jnp.int32, (1, ept), 1)
            valid = pos < e
            src_o = jnp.where(valid, ei[0:1, :], pos & smask)
            dst_o = jnp.where(valid, ei[1:2, :], n + (pos & tmask))
            src_ref[...] = src_o.reshape(1, 1, ept)
            dst_ref[...] = dst_o.reshape(1, 1, ept)

    return pl.pallas_call(
        body,
        grid=(NW,),
        in_specs=[pl.BlockSpec((2, ept), lambda w: (0, w))],
        out_specs=[
            pl.BlockSpec((1, 1, ept), lambda w: (w, 0, 0)),
            pl.BlockSpec((1, 1, ept), lambda w: (w, 0, 0)),
        ],
        out_shape=[
            jax.ShapeDtypeStruct((NW, 1, ept), jnp.int32),
            jax.ShapeDtypeStruct((NW, 1, ept), jnp.int32),
        ],
    )


def _make_deg_kernel(n_pad, ept):
    @functools.partial(
        pl.kernel,
        out_type=jax.ShapeDtypeStruct((NW, n_pad), jnp.float32),
        mesh=_sc_mesh(),
        compiler_params=pltpu.CompilerParams(needs_layout_passes=False),
        scratch_types=[
            pltpu.VMEM((ept,), jnp.int32),
            pltpu.VMEM((n_pad,), jnp.float32),
        ],
    )
    def deg_kernel(dst_hbm, degp_hbm, idx_v, cnt_v):
        cid = lax.axis_index("c")
        sid = lax.axis_index("s")
        wid = cid * NS + sid
        zeros = jnp.zeros((LANES,), jnp.float32)

        @pl.loop(0, n_pad // LANES, unroll=8)
        def _(i):
            cnt_v[pl.ds(i * LANES, LANES)] = zeros

        pltpu.sync_copy(dst_hbm.at[wid, 0], idx_v)
        ones = jnp.ones((LANES,), jnp.float32)

        @pl.loop(0, ept // LANES, unroll=8)
        def _(i):
            idx = idx_v[pl.ds(i * LANES, LANES)]
            plsc.addupdate_scatter(cnt_v, [idx], ones)

        pltpu.sync_copy(cnt_v, degp_hbm.at[wid])

    return deg_kernel


def _make_scatter_kernel(n, d, n_pad, cpt):
    z_rows_per_tile = n_pad // NS
    ept = cpt * CHUNK

    @functools.partial(
        pl.kernel,
        out_type=jax.ShapeDtypeStruct((NC, n_pad, d), jnp.float32),
        mesh=_sc_mesh(),
        scratch_types=[
            pltpu.VMEM((ept,), jnp.int32),
            pltpu.VMEM((2, CHUNK), jnp.int32),
            pltpu.VMEM((CHUNK, d), jnp.float32),
            pltpu.VMEM((CHUNK, d), jnp.float32),
            pltpu.VMEM_SHARED((n_pad, d), jnp.float32),
            pltpu.SemaphoreType.DMA,
            pltpu.SemaphoreType.DMA,
            pltpu.SemaphoreType.DMA,
            pltpu.SemaphoreType.DMA,
        ],
    )
    def scatter_kernel(y_hbm, src_hbm, dst_hbm, z_hbm,
                       src_v, dstr, buf0, buf1, z_sh,
                       gsem0, gsem1, dsem0, dsem1):
        cid = lax.axis_index("c")
        sid = lax.axis_index("s")
        wid = cid * NS + sid

        pltpu.sync_copy(src_hbm.at[wid, 0], src_v)

        zeros = jnp.zeros((LANES,), jnp.float32)

        @pl.loop(0, CHUNK)
        def _(r):
            for k in range(d // LANES):
                buf0[r, pl.ds(k * LANES, LANES)] = zeros

        base = sid * z_rows_per_tile
        for k in range(z_rows_per_tile // CHUNK):
            pltpu.sync_copy(buf0, z_sh.at[pl.ds(base + k * CHUNK, CHUNK)])
        plsc.subcore_barrier()

        bufs = (buf0, buf1)
        gsems = (gsem0, gsem1)
        dsems = (dsem0, dsem1)
        for s in (0, 1):
            pltpu.async_copy(dst_hbm.at[wid, 0, pl.ds(s * CHUNK, CHUNK)],
                             dstr.at[s], dsems[s])
            pltpu.async_copy(y_hbm.at[src_v.at[pl.ds(s * CHUNK, CHUNK)]],
                             bufs[s], gsems[s])

        @pl.loop(0, cpt, step=2)
        def _(jo):
            for b in range(2):
                j = jo + b
                pltpu.make_async_copy(dst_hbm.at[wid, 0, pl.ds(0, CHUNK)],
                                      dstr.at[b], dsems[b]).wait()
                pltpu.make_async_copy(y_hbm.at[src_v.at[pl.ds(0, CHUNK)]],
                                      bufs[b], gsems[b]).wait()
                pltpu.sync_copy(bufs[b], z_sh.at[dstr.at[b]], add=True)

                @pl.when(j + 2 < cpt)
                def _():
                    pltpu.async_copy(
                        dst_hbm.at[wid, 0, pl.ds((j + 2) * CHUNK, CHUNK)],
                        dstr.at[b], dsems[b])
                    pltpu.async_copy(
                        y_hbm.at[src_v.at[pl.ds((j + 2) * CHUNK, CHUNK)]],
                        bufs[b], gsems[b])

        plsc.subcore_barrier()
        pltpu.sync_copy(z_sh.at[pl.ds(base, z_rows_per_tile)],
                        z_hbm.at[cid, pl.ds(base, z_rows_per_tile)])

    return scatter_kernel


def _make_prep_kernel(n, d, n_pad):
    def body(x_ref, w_ref, degp_ref, y_ref, dinv_ref):
        cnt = jnp.sum(degp_ref[...], axis=0)[:n]
        dinv = lax.rsqrt(cnt + 1.0)
        xw = jnp.dot(x_ref[...], w_ref[...], preferred_element_type=jnp.float32)
        y_ref[...] = xw * dinv[:, None]
        dinv_ref[...] = dinv[:, None]

    return pl.pallas_call(
        body,
        grid=(1,),
        in_specs=[
            pl.BlockSpec((n, d), lambda i: (0, 0)),
            pl.BlockSpec((d, d), lambda i: (0, 0)),
            pl.BlockSpec((NW, n_pad), lambda i: (0, 0)),
        ],
        out_specs=[
            pl.BlockSpec((n, d), lambda i: (0, 0)),
            pl.BlockSpec((n, 1), lambda i: (0, 0)),
        ],
        out_shape=[
            jax.ShapeDtypeStruct((n, d), jnp.float32),
            jax.ShapeDtypeStruct((n, 1), jnp.float32),
        ],
    )


def _make_head_kernel(n, d):
    def body(z_ref, y_ref, dinv_ref, bc_ref, w1_ref, b1_ref, w2_ref, b2_ref,
             o_ref):
        zsum = z_ref[0] + z_ref[1] + y_ref[...]
        h = jnp.maximum(zsum * dinv_ref[...] + bc_ref[...], 0.0)
        h = jnp.maximum(
            jnp.dot(h, w1_ref[...], preferred_element_type=jnp.float32)
            + b1_ref[...], 0.0)
        h = jnp.maximum(
            jnp.dot(h, w2_ref[...], preferred_element_type=jnp.float32)
            + b2_ref[...], 0.0)
        o_ref[...] = jax.nn.sigmoid(h)

    return pl.pallas_call(
        body,
        grid=(1,),
        in_specs=[
            pl.BlockSpec((NC, n, d), lambda i: (0, 0, 0)),
            pl.BlockSpec((n, d), lambda i: (0, 0)),
            pl.BlockSpec((n, 1), lambda i: (0, 0)),
            pl.BlockSpec((1, d), lambda i: (0, 0)),
            pl.BlockSpec((d, d), lambda i: (0, 0)),
            pl.BlockSpec((1, d), lambda i: (0, 0)),
            pl.BlockSpec((d, 1), lambda i: (0, 0)),
            pl.BlockSpec((1, 1), lambda i: (0, 0)),
        ],
        out_specs=pl.BlockSpec((n, 1), lambda i: (0, 0)),
        out_shape=jax.ShapeDtypeStruct((n, 1), jnp.float32),
    )


def kernel(x, edge_index, W_conv, b_conv, W_lin1, b_lin1, W_lin2, b_lin2):
    n, d = x.shape
    e = edge_index.shape[1]

    cpt = pl.cdiv(e, NW * CHUNK)
    cpt = cpt + (cpt % 2)
    ept = cpt * CHUNK
    n_pad = ((n + NS * CHUNK - 1) // (NS * CHUNK)) * (NS * CHUNK)
    if n_pad == n:
        n_pad += NS * CHUNK

    srcp, dstp = _make_repack_kernel(e, n, n_pad, ept)(edge_index)
    degp = _make_deg_kernel(n_pad, ept)(dstp)
    y, dinv = _make_prep_kernel(n, d, n_pad)(x, W_conv, degp)
    z = _make_scatter_kernel(n, d, n_pad, cpt)(y, srcp, dstp)
    out = _make_head_kernel(n, d)(
        z, y, dinv, b_conv.reshape(1, d), W_lin1, b_lin1.reshape(1, d),
        W_lin2, b_lin2.reshape(1, 1))
    return out


# speedup vs baseline: 1.0560x; 1.0124x over previous
import functools

import jax
import jax.numpy as jnp
from jax import lax
from jax.experimental import pallas as pl
from jax.experimental.pallas import tpu as pltpu
from jax.experimental.pallas import tpu_sc as plsc

NC = 2
NS = 16
NW = NC * NS
LANES = 16
CHUNK = 128


def _sc_mesh():
    return plsc.VectorSubcoreMesh(
        core_axis_name="c", subcore_axis_name="s", num_cores=NC, num_subcores=NS
    )


def _make_repack_kernel(e, n, n_pad, ept):
    smask = (1 << (n.bit_length() - 1)) - 1
    tmask = (1 << ((n_pad - n).bit_length() - 1)) - 1
    last_full = e // ept

    def body(ei_ref, src_ref, dst_ref):
        w = pl.program_id(0)
        ei = ei_ref[...]

        @pl.when(w < last_full)
        def _():
            src_ref[...] = ei[0:1, :].reshape(1, 1, ept)
            dst_ref[...] = ei[1:2, :].reshape(1, 1, ept)

        @pl.when(w >= last_full)
        def _():
            pos = w * ept + lax.broadcasted_iota(jnp.int32, (1, ept), 1)
            valid = pos < e
            src_o = jnp.where(valid, ei[0:1, :], pos & smask)
            dst_o = jnp.where(valid, ei[1:2, :], n + (pos & tmask))
            src_ref[...] = src_o.reshape(1, 1, ept)
            dst_ref[...] = dst_o.reshape(1, 1, ept)

    return pl.pallas_call(
        body,
        grid=(NW,),
        in_specs=[pl.BlockSpec((2, ept), lambda w: (0, w))],
        out_specs=[
            pl.BlockSpec((1, 1, ept), lambda w: (w, 0, 0)),
            pl.BlockSpec((1, 1, ept), lambda w: (w, 0, 0)),
        ],
        out_shape=[
            jax.ShapeDtypeStruct((NW, 1, ept), jnp.int32),
            jax.ShapeDtypeStruct((NW, 1, ept), jnp.int32),
        ],
    )


def _make_deg_kernel(n_pad, ept):
    @functools.partial(
        pl.kernel,
        out_type=jax.ShapeDtypeStruct((NW, n_pad), jnp.float32),
        mesh=_sc_mesh(),
        compiler_params=pltpu.CompilerParams(needs_layout_passes=False),
        scratch_types=[
            pltpu.VMEM((ept,), jnp.int32),
            pltpu.VMEM((n_pad,), jnp.float32),
        ],
    )
    def deg_kernel(dst_hbm, degp_hbm, idx_v, cnt_v):
        cid = lax.axis_index("c")
        sid = lax.axis_index("s")
        wid = cid * NS + sid
        zeros = jnp.zeros((LANES,), jnp.float32)

        @pl.loop(0, n_pad // LANES, unroll=8)
        def _(i):
            cnt_v[pl.ds(i * LANES, LANES)] = zeros

        pltpu.sync_copy(dst_hbm.at[wid, 0], idx_v)
        ones = jnp.ones((LANES,), jnp.float32)

        @pl.loop(0, ept // LANES, unroll=8)
        def _(i):
            idx = idx_v[pl.ds(i * LANES, LANES)]
            plsc.addupdate_scatter(cnt_v, [idx], ones)

        pltpu.sync_copy(cnt_v, degp_hbm.at[wid])

    return deg_kernel


def _make_scatter_kernel(n, d, n_pad, cpt):
    z_rows_per_tile = n_pad // NS
    ept = cpt * CHUNK

    @functools.partial(
        pl.kernel,
        out_type=jax.ShapeDtypeStruct((NC, n_pad, d), jnp.float32),
        mesh=_sc_mesh(),
        scratch_types=[
            pltpu.VMEM((ept,), jnp.int32),
            pltpu.VMEM((2, CHUNK), jnp.int32),
            pltpu.VMEM((CHUNK, d), jnp.float32),
            pltpu.VMEM((CHUNK, d), jnp.float32),
            pltpu.VMEM((32, d), jnp.float32),
            pltpu.VMEM_SHARED((n_pad, d), jnp.float32),
            pltpu.SemaphoreType.DMA,
            pltpu.SemaphoreType.DMA,
            pltpu.SemaphoreType.DMA,
            pltpu.SemaphoreType.DMA,
        ],
    )
    def scatter_kernel(y_hbm, src_hbm, dst_hbm, z_hbm,
                       src_v, dstr, buf0, buf1, zbuf, z_sh,
                       gsem0, gsem1, dsem0, dsem1):
        cid = lax.axis_index("c")
        sid = lax.axis_index("s")
        wid = cid * NS + sid

        pltpu.sync_copy(src_hbm.at[wid, 0], src_v)

        bufs = (buf0, buf1)
        gsems = (gsem0, gsem1)
        dsems = (dsem0, dsem1)
        for s in (0, 1):
            pltpu.async_copy(dst_hbm.at[wid, 0, pl.ds(s * CHUNK, CHUNK)],
                             dstr.at[s], dsems[s])
            pltpu.async_copy(y_hbm.at[src_v.at[pl.ds(s * CHUNK, CHUNK)]],
                             bufs[s], gsems[s])

        zeros = jnp.zeros((LANES,), jnp.float32)

        @pl.loop(0, 32, unroll=4)
        def _(r):
            for k in range(d // LANES):
                zbuf[r, pl.ds(k * LANES, LANES)] = zeros

        base = sid * z_rows_per_tile
        for k in range(z_rows_per_tile // 32):
            pltpu.sync_copy(zbuf, z_sh.at[pl.ds(base + k * 32, 32)])
        plsc.subcore_barrier()

        @pl.loop(0, cpt, step=2)
        def _(jo):
            for b in range(2):
                j = jo + b
                pltpu.make_async_copy(dst_hbm.at[wid, 0, pl.ds(0, CHUNK)],
                                      dstr.at[b], dsems[b]).wait()
                pltpu.make_async_copy(y_hbm.at[src_v.at[pl.ds(0, CHUNK)]],
                                      bufs[b], gsems[b]).wait()
                pltpu.sync_copy(bufs[b], z_sh.at[dstr.at[b]], add=True)

                @pl.when(j + 2 < cpt)
                def _():
                    pltpu.async_copy(
                        dst_hbm.at[wid, 0, pl.ds((j + 2) * CHUNK, CHUNK)],
                        dstr.at[b], dsems[b])
                    pltpu.async_copy(
                        y_hbm.at[src_v.at[pl.ds((j + 2) * CHUNK, CHUNK)]],
                        bufs[b], gsems[b])

        plsc.subcore_barrier()
        pltpu.sync_copy(z_sh.at[pl.ds(base, z_rows_per_tile)],
                        z_hbm.at[cid, pl.ds(base, z_rows_per_tile)])

    return scatter_kernel


def _make_prep_kernel(n, d, n_pad):
    def body(x_ref, w_ref, degp_ref, y_ref, dinv_ref):
        cnt = jnp.sum(degp_ref[...], axis=0)[:n]
        dinv = lax.rsqrt(cnt + 1.0)
        xw = jnp.dot(x_ref[...], w_ref[...], preferred_element_type=jnp.float32)
        y_ref[...] = xw * dinv[:, None]
        dinv_ref[...] = dinv[:, None]

    return pl.pallas_call(
        body,
        grid=(1,),
        in_specs=[
            pl.BlockSpec((n, d), lambda i: (0, 0)),
            pl.BlockSpec((d, d), lambda i: (0, 0)),
            pl.BlockSpec((NW, n_pad), lambda i: (0, 0)),
        ],
        out_specs=[
            pl.BlockSpec((n, d), lambda i: (0, 0)),
            pl.BlockSpec((n, 1), lambda i: (0, 0)),
        ],
        out_shape=[
            jax.ShapeDtypeStruct((n, d), jnp.float32),
            jax.ShapeDtypeStruct((n, 1), jnp.float32),
        ],
    )


def _make_head_kernel(n, d):
    def body(z_ref, y_ref, dinv_ref, bc_ref, w1_ref, b1_ref, w2_ref, b2_ref,
             o_ref):
        zsum = z_ref[0] + z_ref[1] + y_ref[...]
        h = jnp.maximum(zsum * dinv_ref[...] + bc_ref[...], 0.0)
        h = jnp.maximum(
            jnp.dot(h, w1_ref[...], preferred_element_type=jnp.float32)
            + b1_ref[...], 0.0)
        h = jnp.maximum(
            jnp.dot(h, w2_ref[...], preferred_element_type=jnp.float32)
            + b2_ref[...], 0.0)
        o_ref[...] = jax.nn.sigmoid(h)

    return pl.pallas_call(
        body,
        grid=(1,),
        in_specs=[
            pl.BlockSpec((NC, n, d), lambda i: (0, 0, 0)),
            pl.BlockSpec((n, d), lambda i: (0, 0)),
            pl.BlockSpec((n, 1), lambda i: (0, 0)),
            pl.BlockSpec((1, d), lambda i: (0, 0)),
            pl.BlockSpec((d, d), lambda i: (0, 0)),
            pl.BlockSpec((1, d), lambda i: (0, 0)),
            pl.BlockSpec((d, 1), lambda i: (0, 0)),
            pl.BlockSpec((1, 1), lambda i: (0, 0)),
        ],
        out_specs=pl.BlockSpec((n, 1), lambda i: (0, 0)),
        out_shape=jax.ShapeDtypeStruct((n, 1), jnp.float32),
    )


def kernel(x, edge_index, W_conv, b_conv, W_lin1, b_lin1, W_lin2, b_lin2):
    n, d = x.shape
    e = edge_index.shape[1]

    cpt = pl.cdiv(e, NW * CHUNK)
    cpt = cpt + (cpt % 2)
    ept = cpt * CHUNK
    n_pad = ((n + NS * CHUNK - 1) // (NS * CHUNK)) * (NS * CHUNK)
    if n_pad == n:
        n_pad += NS * CHUNK

    srcp, dstp = _make_repack_kernel(e, n, n_pad, ept)(edge_index)
    degp = _make_deg_kernel(n_pad, ept)(dstp)
    y, dinv = _make_prep_kernel(n, d, n_pad)(x, W_conv, degp)
    z = _make_scatter_kernel(n, d, n_pad, cpt)(y, srcp, dstp)
    out = _make_head_kernel(n, d)(
        z, y, dinv, b_conv.reshape(1, d), W_lin1, b_lin1.reshape(1, d),
        W_lin2, b_lin2.reshape(1, 1))
    return out


# speedup vs baseline: 1.1577x; 1.0963x over previous
import functools

import jax
import jax.numpy as jnp
from jax import lax
from jax.experimental import pallas as pl
from jax.experimental.pallas import tpu as pltpu
from jax.experimental.pallas import tpu_sc as plsc

NC = 2
NS = 16
NW = NC * NS
LANES = 16
CHUNK = 128


def _sc_mesh():
    return plsc.VectorSubcoreMesh(
        core_axis_name="c", subcore_axis_name="s", num_cores=NC, num_subcores=NS
    )


def _make_repack_kernel(e, n, n_pad, ept):
    smask = (1 << (n.bit_length() - 1)) - 1
    tmask = (1 << ((n_pad - n).bit_length() - 1)) - 1

    def body(ei_ref, src_ref, dst_ref):
        ei = ei_ref[...]
        pos = (NW - 1) * ept + lax.broadcasted_iota(jnp.int32, (1, ept), 1)
        valid = pos < e
        src_ref[...] = jnp.where(valid, ei[0:1, :], pos & smask)
        dst_ref[...] = jnp.where(valid, ei[1:2, :], n + (pos & tmask))

    return pl.pallas_call(
        body,
        grid=(1,),
        in_specs=[pl.BlockSpec((2, ept), lambda i: (0, NW - 1))],
        out_specs=[
            pl.BlockSpec((1, ept), lambda i: (0, 0)),
            pl.BlockSpec((1, ept), lambda i: (0, 0)),
        ],
        out_shape=[
            jax.ShapeDtypeStruct((1, ept), jnp.int32),
            jax.ShapeDtypeStruct((1, ept), jnp.int32),
        ],
    )


def _make_deg_kernel(n_pad, ept):
    @functools.partial(
        pl.kernel,
        out_type=jax.ShapeDtypeStruct((NW, n_pad), jnp.float32),
        mesh=_sc_mesh(),
        compiler_params=pltpu.CompilerParams(needs_layout_passes=False),
        scratch_types=[
            pltpu.VMEM((ept,), jnp.int32),
            pltpu.VMEM((n_pad,), jnp.float32),
        ],
    )
    def deg_kernel(ei_hbm, tdst_hbm, degp_hbm, idx_v, cnt_v):
        cid = lax.axis_index("c")
        sid = lax.axis_index("s")
        wid = cid * NS + sid
        zeros = jnp.zeros((LANES,), jnp.float32)

        @pl.loop(0, n_pad // LANES, unroll=8)
        def _(i):
            cnt_v[pl.ds(i * LANES, LANES)] = zeros

        @pl.when(wid < NW - 1)
        def _():
            pltpu.sync_copy(ei_hbm.at[1, pl.ds(wid * ept, ept)], idx_v)

        @pl.when(wid == NW - 1)
        def _():
            pltpu.sync_copy(tdst_hbm.at[0], idx_v)

        ones = jnp.ones((LANES,), jnp.float32)

        @pl.loop(0, ept // LANES, unroll=8)
        def _(i):
            idx = idx_v[pl.ds(i * LANES, LANES)]
            plsc.addupdate_scatter(cnt_v, [idx], ones)

        pltpu.sync_copy(cnt_v, degp_hbm.at[wid])

    return deg_kernel


def _make_scatter_kernel(n, d, n_pad, cpt):
    z_rows_per_tile = n_pad // NS
    ept = cpt * CHUNK

    @functools.partial(
        pl.kernel,
        out_type=jax.ShapeDtypeStruct((NC, n_pad, d), jnp.float32),
        mesh=_sc_mesh(),
        scratch_types=[
            pltpu.VMEM((ept,), jnp.int32),
            pltpu.VMEM((2, CHUNK), jnp.int32),
            pltpu.VMEM((CHUNK, d), jnp.float32),
            pltpu.VMEM((CHUNK, d), jnp.float32),
            pltpu.VMEM((32, d), jnp.float32),
            pltpu.VMEM_SHARED((n_pad, d), jnp.float32),
            pltpu.SemaphoreType.DMA,
            pltpu.SemaphoreType.DMA,
            pltpu.SemaphoreType.DMA,
            pltpu.SemaphoreType.DMA,
        ],
    )
    def scatter_kernel(y_hbm, ei_hbm, tsrc_hbm, tdst_hbm, z_hbm,
                       src_v, dstr, buf0, buf1, zbuf, z_sh,
                       gsem0, gsem1, dsem0, dsem1):
        cid = lax.axis_index("c")
        sid = lax.axis_index("s")
        wid = cid * NS + sid
        is_tail = wid == NW - 1

        @pl.when(jnp.logical_not(is_tail))
        def _():
            pltpu.sync_copy(ei_hbm.at[0, pl.ds(wid * ept, ept)], src_v)

        @pl.when(is_tail)
        def _():
            pltpu.sync_copy(tsrc_hbm.at[0], src_v)

        def issue_dst(j, s):
            @pl.when(jnp.logical_not(is_tail))
            def _():
                pltpu.async_copy(
                    ei_hbm.at[1, pl.ds(wid * ept + j * CHUNK, CHUNK)],
                    dstr.at[s], dsems[s])

            @pl.when(is_tail)
            def _():
                pltpu.async_copy(tdst_hbm.at[0, pl.ds(j * CHUNK, CHUNK)],
                                 dstr.at[s], dsems[s])

        bufs = (buf0, buf1)
        gsems = (gsem0, gsem1)
        dsems = (dsem0, dsem1)
        for s in (0, 1):
            issue_dst(s, s)
            pltpu.async_copy(y_hbm.at[src_v.at[pl.ds(s * CHUNK, CHUNK)]],
                             bufs[s], gsems[s])

        zeros = jnp.zeros((LANES,), jnp.float32)

        @pl.loop(0, 32, unroll=4)
        def _(r):
            for k in range(d // LANES):
                zbuf[r, pl.ds(k * LANES, LANES)] = zeros

        base = sid * z_rows_per_tile
        for k in range(z_rows_per_tile // 32):
            pltpu.sync_copy(zbuf, z_sh.at[pl.ds(base + k * 32, 32)])
        plsc.subcore_barrier()

        @pl.loop(0, cpt, step=2)
        def _(jo):
            for b in range(2):
                j = jo + b
                pltpu.make_async_copy(ei_hbm.at[1, pl.ds(0, CHUNK)],
                                      dstr.at[b], dsems[b]).wait()
                pltpu.make_async_copy(y_hbm.at[src_v.at[pl.ds(0, CHUNK)]],
                                      bufs[b], gsems[b]).wait()
                pltpu.sync_copy(bufs[b], z_sh.at[dstr.at[b]], add=True)

                @pl.when(j + 2 < cpt)
                def _():
                    issue_dst(j + 2, b)
                    pltpu.async_copy(
                        y_hbm.at[src_v.at[pl.ds((j + 2) * CHUNK, CHUNK)]],
                        bufs[b], gsems[b])

        plsc.subcore_barrier()
        pltpu.sync_copy(z_sh.at[pl.ds(base, z_rows_per_tile)],
                        z_hbm.at[cid, pl.ds(base, z_rows_per_tile)])

    return scatter_kernel


def _make_prep_kernel(n, d, n_pad):
    def body(x_ref, w_ref, degp_ref, y_ref, dinv_ref):
        cnt = jnp.sum(degp_ref[...], axis=0)[:n]
        dinv = lax.rsqrt(cnt + 1.0)
        xw = jnp.dot(x_ref[...], w_ref[...], preferred_element_type=jnp.float32)
        y_ref[...] = xw * dinv[:, None]
        dinv_ref[...] = dinv[:, None]

    return pl.pallas_call(
        body,
        grid=(1,),
        in_specs=[
            pl.BlockSpec((n, d), lambda i: (0, 0)),
            pl.BlockSpec((d, d), lambda i: (0, 0)),
            pl.BlockSpec((NW, n_pad), lambda i: (0, 0)),
        ],
        out_specs=[
            pl.BlockSpec((n, d), lambda i: (0, 0)),
            pl.BlockSpec((n, 1), lambda i: (0, 0)),
        ],
        out_shape=[
            jax.ShapeDtypeStruct((n, d), jnp.float32),
            jax.ShapeDtypeStruct((n, 1), jnp.float32),
        ],
    )


def _make_head_kernel(n, d):
    def body(z_ref, y_ref, dinv_ref, bc_ref, w1_ref, b1_ref, w2_ref, b2_ref,
             o_ref):
        zsum = z_ref[0] + z_ref[1] + y_ref[...]
        h = jnp.maximum(zsum * dinv_ref[...] + bc_ref[...], 0.0)
        h = jnp.maximum(
            jnp.dot(h, w1_ref[...], preferred_element_type=jnp.float32)
            + b1_ref[...], 0.0)
        h = jnp.maximum(
            jnp.dot(h, w2_ref[...], preferred_element_type=jnp.float32)
            + b2_ref[...], 0.0)
        o_ref[...] = jax.nn.sigmoid(h)

    return pl.pallas_call(
        body,
        grid=(1,),
        in_specs=[
            pl.BlockSpec((NC, n, d), lambda i: (0, 0, 0)),
            pl.BlockSpec((n, d), lambda i: (0, 0)),
            pl.BlockSpec((n, 1), lambda i: (0, 0)),
            pl.BlockSpec((1, d), lambda i: (0, 0)),
            pl.BlockSpec((d, d), lambda i: (0, 0)),
            pl.BlockSpec((1, d), lambda i: (0, 0)),
            pl.BlockSpec((d, 1), lambda i: (0, 0)),
            pl.BlockSpec((1, 1), lambda i: (0, 0)),
        ],
        out_specs=pl.BlockSpec((n, 1), lambda i: (0, 0)),
        out_shape=jax.ShapeDtypeStruct((n, 1), jnp.float32),
    )


def kernel(x, edge_index, W_conv, b_conv, W_lin1, b_lin1, W_lin2, b_lin2):
    n, d = x.shape
    e = edge_index.shape[1]

    cpt = pl.cdiv(e, NW * CHUNK)
    cpt = cpt + (cpt % 2)
    ept = cpt * CHUNK
    n_pad = ((n + NS * CHUNK - 1) // (NS * CHUNK)) * (NS * CHUNK)
    if n_pad == n:
        n_pad += NS * CHUNK

    tsrc, tdst = _make_repack_kernel(e, n, n_pad, ept)(edge_index)
    degp = _make_deg_kernel(n_pad, ept)(edge_index, tdst)
    y, dinv = _make_prep_kernel(n, d, n_pad)(x, W_conv, degp)
    z = _make_scatter_kernel(n, d, n_pad, cpt)(y, edge_index, tsrc, tdst)
    out = _make_head_kernel(n, d)(
        z, y, dinv, b_conv.reshape(1, d), W_lin1, b_lin1.reshape(1, d),
        W_lin2, b_lin2.reshape(1, 1))
    return out
